# Initial kernel scaffold; baseline (speedup 1.0000x reference)
import jax
import jax.numpy as jnp
from jax import lax
from jax.experimental import pallas as pl
from jax.experimental.pallas import tpu as pltpu
from jax.experimental.pallas import tpu_sc as plsc

_B, _C, _L, _N = 16, 512, 4096, 1024
_GAMMA = 1.0
_NW = 32
_CW = _C // _NW
_LANES = 16
_NG = _N // _LANES


def _sampler_body(inp, pt, off, out, pt_v, off_v, i0_v, w1_v, inp_buf, out_buf):
    cid = lax.axis_index("c")
    sid = lax.axis_index("s")
    wid = sid * 2 + cid
    c0 = wid * _CW

    def per_batch(b, _):
        pltpu.sync_copy(pt.at[b], pt_v)
        pltpu.sync_copy(off.at[b], off_v)

        def idx_body(j, _):
            sl = pl.ds(j * _LANES, _LANES)
            loc = pt_v[sl] + _GAMMA * off_v[sl]
            loc = jnp.minimum(jnp.maximum(loc, 0.0), float(_L - 1))
            i0 = loc.astype(jnp.int32)
            i0_v[sl] = i0
            w1_v[sl] = loc - i0.astype(jnp.float32)
            return 0

        lax.fori_loop(0, _NG, idx_body, 0)

        pltpu.sync_copy(inp.at[b, pl.ds(c0, _CW)], inp_buf)

        def grp_body(g, _):
            n_base = g * _LANES
            sl = pl.ds(n_base, _LANES)
            i0 = i0_v[sl]
            w1 = w1_v[sl]
            i1 = jnp.minimum(i0 + 1, _L - 1)
            w0 = 1.0 - w1
            n_idx = n_base + lax.iota(jnp.int32, _LANES)
            for c in range(_CW):
                c_idx = jnp.full((_LANES,), c, jnp.int32)
                v0 = plsc.load_gather(inp_buf, [c_idx, i0])
                v1 = plsc.load_gather(inp_buf, [c_idx, i1])
                r = w0 * v0 + w1 * v1
                plsc.store_scatter(out_buf, [n_idx, c_idx], r)
            return 0

        lax.fori_loop(0, _NG, grp_body, 0)

        pltpu.sync_copy(out_buf, out.at[b, :, pl.ds(c0, _CW)])
        return 0

    lax.fori_loop(0, _B, per_batch, 0)


def kernel(input, point, offset):
    pt = point[:, :, 0]
    off = offset[:, :, 0]
    mesh = plsc.VectorSubcoreMesh(core_axis_name="c", subcore_axis_name="s")
    f = pl.kernel(
        _sampler_body,
        out_type=jax.ShapeDtypeStruct((_B, _N, _C), jnp.float32),
        mesh=mesh,
        scratch_types=[
            pltpu.VMEM((_N,), jnp.float32),
            pltpu.VMEM((_N,), jnp.float32),
            pltpu.VMEM((_N,), jnp.int32),
            pltpu.VMEM((_N,), jnp.float32),
            pltpu.VMEM((_CW, _L), jnp.float32),
            pltpu.VMEM((_N, _CW), jnp.float32),
        ],
    )
    return f(input, pt, off)


# speedup vs baseline: 2.2381x; 2.2381x over previous
import jax
import jax.numpy as jnp
from jax import lax
from jax.experimental import pallas as pl
from jax.experimental.pallas import tpu as pltpu
from jax.experimental.pallas import tpu_sc as plsc

_B, _C, _L, _N = 16, 512, 4096, 1024
_GAMMA = 1.0
_NW = 32
_CW = _C // _NW
_LANES = 16
_NG = _N // _LANES


def _sampler_body(inp, pt, off, out, pt_v, off_v, i0_v, w1_v, inp_buf, out_buf):
    cid = lax.axis_index("c")
    sid = lax.axis_index("s")
    wid = sid * 2 + cid
    c0 = wid * _CW

    def per_batch(b, _):
        pltpu.sync_copy(pt.at[b], pt_v)
        pltpu.sync_copy(off.at[b], off_v)

        def idx_body(j, _):
            sl = pl.ds(j * _LANES, _LANES)
            loc = pt_v[sl] + _GAMMA * off_v[sl]
            loc = jnp.minimum(jnp.maximum(loc, 0.0), float(_L - 1))
            i0 = loc.astype(jnp.int32)
            i0_v[sl] = i0
            w1_v[sl] = loc - i0.astype(jnp.float32)
            return 0

        lax.fori_loop(0, _NG, idx_body, 0)

        pltpu.sync_copy(inp.at[b, pl.ds(c0, _CW)], inp_buf)

        def grp_body(g, _):
            n_base = g * _LANES
            sl = pl.ds(n_base, _LANES)
            i0 = i0_v[sl]
            w1 = w1_v[sl]
            i1 = jnp.minimum(i0 + 1, _L - 1)
            w0 = 1.0 - w1
            n_idx = n_base + lax.iota(jnp.int32, _LANES)
            for c in range(_CW):
                c_idx = jnp.full((_LANES,), c, jnp.int32)
                v0 = plsc.load_gather(inp_buf, [c_idx, i0])
                v1 = plsc.load_gather(inp_buf, [c_idx, i1])
                r = w0 * v0 + w1 * v1
                plsc.store_scatter(out_buf, [n_idx, c_idx], r)
            return 0

        lax.fori_loop(0, _NG, grp_body, 0)

        pltpu.sync_copy(out_buf, out.at[b, :, pl.ds(c0, _CW)])
        return 0

    lax.fori_loop(0, _B, per_batch, 0)


def kernel(input, point, offset):
    pt = point[:, :, 0]
    off = offset[:, :, 0]
    mesh = plsc.VectorSubcoreMesh(core_axis_name="c", subcore_axis_name="s")
    f = pl.kernel(
        _sampler_body,
        out_type=jax.ShapeDtypeStruct((_B, _N, _C), jnp.float32),
        mesh=mesh,
        scratch_types=[
            pltpu.VMEM((_N,), jnp.float32),
            pltpu.VMEM((_N,), jnp.float32),
            pltpu.VMEM((_N,), jnp.int32),
            pltpu.VMEM((_N,), jnp.float32),
            pltpu.VMEM((_CW, _L), jnp.float32),
            pltpu.VMEM((_N, _CW), jnp.float32),
        ],
        compiler_params=pltpu.CompilerParams(
            use_tc_tiling_on_sc=False, needs_layout_passes=False
        ),
    )
    return f(input, pt, off)
